# Initial kernel scaffold; baseline (speedup 1.0000x reference)
import jax
import jax.numpy as jnp
from jax.experimental import pallas as pl


def kernel(att_logits, ctc_logits, targets, target_lengths):
    raise NotImplementedError("write your pallas kernel here")


# speedup vs baseline: 2.6118x; 2.6118x over previous
import functools

import jax
import jax.numpy as jnp
from jax import lax
from jax.experimental import pallas as pl
from jax.experimental.pallas import tpu as pltpu

PAD_ID = 0
BLANK = 1
ALPHA = 0.8
SMOOTHING = 0.1
NEG_INF = -1e30

TC_CHUNK = 32
B_CHUNK = 8


def _att_body(x_ref, t_ref, out_ref, acc_ref, *, v, c1, c2, n_rows):
    i = pl.program_id(0)
    x = x_ref[...]
    tgt = t_ref[...]
    rowsum = jnp.sum(x, axis=2)
    iota_v = lax.broadcasted_iota(jnp.int32, x.shape, 2)
    tval = jnp.sum(jnp.where(iota_v == tgt[:, :, None], x, 0.0), axis=2)
    contrib = jnp.where(tgt != PAD_ID, -(c1 * rowsum + c2 * tval), 0.0)
    part = jnp.sum(contrib)

    @pl.when(i == 0)
    def _():
        acc_ref[0, 0] = part

    @pl.when(i > 0)
    def _():
        acc_ref[0, 0] = acc_ref[0, 0] + part

    @pl.when(i == pl.num_programs(0) - 1)
    def _():
        out_ref[...] = jnp.full((1, 1), acc_ref[0, 0] * (1.0 / n_rows),
                                jnp.float32)


def _att_loss_call(att_logits, targets):
    b, s, v = att_logits.shape
    c1 = SMOOTHING / (v - 1)
    c2 = (1.0 - SMOOTHING) - c1
    grid = b // B_CHUNK
    return pl.pallas_call(
        functools.partial(_att_body, v=v, c1=c1, c2=c2, n_rows=b * s),
        grid=(grid,),
        in_specs=[
            pl.BlockSpec((B_CHUNK, s, v), lambda i: (i, 0, 0)),
            pl.BlockSpec((B_CHUNK, s), lambda i: (i, 0)),
        ],
        out_specs=pl.BlockSpec((1, 1), lambda i: (0, 0)),
        out_shape=jax.ShapeDtypeStruct((1, 1), jnp.float32),
        scratch_shapes=[pltpu.SMEM((1, 1), jnp.float32)],
        interpret=False,
    )(att_logits, targets)


def _ctc_body(x_ref, t_ref, len_ref, out_ref,
              e_ref, p_ref, ae_ref, ao_ref, m_ref, ohe_ref, oho_ref,
              *, b, s, v, t_total):
    ncol = s + 1
    nchunk = t_total // TC_CHUNK
    i = pl.program_id(0)

    @pl.when(i == 0)
    def _init():
        tgt = t_ref[...]
        iota_v = lax.broadcasted_iota(jnp.int32, (v, ncol), 0)
        for bb in range(b):
            ext = jnp.concatenate(
                [jnp.full((1, 1), BLANK, jnp.int32), tgt[bb].reshape(1, s)],
                axis=1)
            e_ref[bb] = jnp.where(iota_v == ext, 1.0, 0.0)
        prev = jnp.concatenate([tgt[:, :1], tgt[:, : s - 1]], axis=1)
        m_ref[...] = ((tgt != prev) & (tgt != BLANK)).astype(jnp.float32)
        lens = len_ref[...]
        iota_e = lax.broadcasted_iota(jnp.int32, (b, ncol), 1)
        iota_o = lax.broadcasted_iota(jnp.int32, (b, s), 1)
        ohe_ref[...] = (iota_e == lens).astype(jnp.float32)
        oho_ref[...] = (iota_o == lens - 1).astype(jnp.float32)
        ae_ref[...] = jnp.zeros((b, ncol), jnp.float32)
        ao_ref[...] = jnp.zeros((b, s), jnp.float32)

    for bb in range(b):
        p_ref[bb] = jnp.dot(x_ref[bb], e_ref[bb],
                            preferred_element_type=jnp.float32)

    @pl.when(i == 0)
    def _init_state():
        row0 = p_ref[:, 0, :]
        iota = lax.broadcasted_iota(jnp.int32, (b, ncol), 1)
        ae_ref[...] = jnp.where(iota == 0, row0, NEG_INF)
        ao_ref[...] = jnp.where(iota[:, :s] == 0, row0[:, 1:], NEG_INF)

    skip = m_ref[...] > 0.5

    def step(t, carry):
        row = p_ref[:, t, :]
        lpb = row[:, :1]
        lpo = row[:, 1:]
        ae = ae_ref[...]
        ao = ao_ref[...]
        ao_pad = jnp.concatenate(
            [jnp.full((b, 1), NEG_INF, jnp.float32), ao], axis=1)
        m1 = jnp.maximum(ae, ao_pad)
        new_ae = m1 + jnp.log1p(jnp.exp(-jnp.abs(ae - ao_pad))) + lpb
        sk = jnp.where(skip, ao_pad[:, :s], NEG_INF)
        ae_s = ae[:, :s]
        m2 = jnp.maximum(jnp.maximum(ao, ae_s), sk)
        new_ao = (m2 + jnp.log(jnp.exp(ao - m2) + jnp.exp(ae_s - m2)
                               + jnp.exp(sk - m2)) + lpo)
        ae_ref[...] = new_ae
        ao_ref[...] = new_ao
        return carry

    start_t = jnp.where(i == 0, 1, 0)
    lax.fori_loop(start_t, TC_CHUNK, step, 0)

    @pl.when(i == nchunk - 1)
    def _final():
        ae = ae_ref[...]
        ao = ao_ref[...]
        ohe = ohe_ref[...] > 0.5
        oho = oho_ref[...] > 0.5
        l_last = jnp.sum(jnp.where(ohe, ae, 0.0), axis=1, keepdims=True)
        l_prev = jnp.sum(jnp.where(oho, ao, 0.0), axis=1, keepdims=True)
        mm = jnp.maximum(l_last, l_prev)
        nll = -(mm + jnp.log(jnp.exp(l_last - mm) + jnp.exp(l_prev - mm)))
        nll = jnp.where(jnp.isinf(nll) | (nll >= 1e29), 0.0, nll)
        lens = jnp.maximum(len_ref[...], 1).astype(jnp.float32)
        out_ref[...] = jnp.sum(nll / lens, keepdims=True).reshape(1, 1) * (1.0 / b)


def _ctc_call(ctc_logits, targets, lengths2d):
    b, t_total, v = ctc_logits.shape
    s = targets.shape[1]
    ncol = s + 1
    nchunk = t_total // TC_CHUNK
    return pl.pallas_call(
        functools.partial(_ctc_body, b=b, s=s, v=v, t_total=t_total),
        grid=(nchunk,),
        in_specs=[
            pl.BlockSpec((b, TC_CHUNK, v), lambda i: (0, i, 0)),
            pl.BlockSpec((b, s), lambda i: (0, 0)),
            pl.BlockSpec((b, 1), lambda i: (0, 0)),
        ],
        out_specs=pl.BlockSpec((1, 1), lambda i: (0, 0)),
        out_shape=jax.ShapeDtypeStruct((1, 1), jnp.float32),
        scratch_shapes=[
            pltpu.VMEM((b, v, ncol), jnp.float32),
            pltpu.VMEM((b, TC_CHUNK, ncol), jnp.float32),
            pltpu.VMEM((b, ncol), jnp.float32),
            pltpu.VMEM((b, s), jnp.float32),
            pltpu.VMEM((b, s), jnp.float32),
            pltpu.VMEM((b, ncol), jnp.float32),
            pltpu.VMEM((b, s), jnp.float32),
        ],
        interpret=False,
    )(ctc_logits, targets, lengths2d)


@jax.jit
def kernel(att_logits, ctc_logits, targets, target_lengths):
    b = att_logits.shape[0]
    att = _att_loss_call(att_logits, targets)[0, 0]
    ctc = _ctc_call(ctc_logits, targets, target_lengths.reshape(b, 1))[0, 0]
    return ALPHA * att + (1.0 - ALPHA) * ctc


# speedup vs baseline: 2.7519x; 1.0536x over previous
import functools

import jax
import jax.numpy as jnp
from jax import lax
from jax.experimental import pallas as pl
from jax.experimental.pallas import tpu as pltpu

PAD_ID = 0
BLANK = 1
ALPHA = 0.8
SMOOTHING = 0.1
NEG_INF = -1e30

TC_CHUNK = 32
B_CHUNK = 8


def _att_body(x_ref, t_ref, out_ref, acc_ref, *, v, c1, c2, n_rows):
    i = pl.program_id(0)
    x = x_ref[...]
    tgt = t_ref[...]
    rowsum = jnp.sum(x, axis=2)
    iota_v = lax.broadcasted_iota(jnp.int32, x.shape, 2)
    tval = jnp.sum(jnp.where(iota_v == tgt[:, :, None], x, 0.0), axis=2)
    contrib = jnp.where(tgt != PAD_ID, -(c1 * rowsum + c2 * tval), 0.0)
    part = jnp.sum(contrib)

    @pl.when(i == 0)
    def _():
        acc_ref[0, 0] = part

    @pl.when(i > 0)
    def _():
        acc_ref[0, 0] = acc_ref[0, 0] + part

    @pl.when(i == pl.num_programs(0) - 1)
    def _():
        out_ref[...] = jnp.full((1, 1), acc_ref[0, 0] * (1.0 / n_rows),
                                jnp.float32)


def _att_loss_call(att_logits, targets):
    b, s, v = att_logits.shape
    c1 = SMOOTHING / (v - 1)
    c2 = (1.0 - SMOOTHING) - c1
    grid = b // B_CHUNK
    return pl.pallas_call(
        functools.partial(_att_body, v=v, c1=c1, c2=c2, n_rows=b * s),
        grid=(grid,),
        in_specs=[
            pl.BlockSpec((B_CHUNK, s, v), lambda i: (i, 0, 0)),
            pl.BlockSpec((B_CHUNK, s), lambda i: (i, 0)),
        ],
        out_specs=pl.BlockSpec((1, 1), lambda i: (0, 0)),
        out_shape=jax.ShapeDtypeStruct((1, 1), jnp.float32),
        scratch_shapes=[pltpu.SMEM((1, 1), jnp.float32)],
        interpret=False,
    )(att_logits, targets)


def _ctc_body(x_ref, t_ref, len_ref, out_ref,
              e_ref, p_ref, ae_ref, ao_ref, m_ref, ohe_ref, oho_ref,
              *, b, s, v, t_total):
    ncol = s + 1
    nchunk = t_total // TC_CHUNK
    i = pl.program_id(0)

    @pl.when(i == 0)
    def _init():
        tgt = t_ref[...]
        iota_v = lax.broadcasted_iota(jnp.int32, (v, ncol), 0)
        for bb in range(b):
            ext = jnp.concatenate(
                [jnp.full((1, 1), BLANK, jnp.int32), tgt[bb].reshape(1, s)],
                axis=1)
            e_ref[bb] = jnp.where(iota_v == ext, 1.0, 0.0)
        prev = jnp.concatenate([tgt[:, :1], tgt[:, : s - 1]], axis=1)
        m_ref[...] = ((tgt != prev) & (tgt != BLANK)).astype(jnp.float32)
        lens = len_ref[...]
        iota_e = lax.broadcasted_iota(jnp.int32, (b, ncol), 1)
        iota_o = lax.broadcasted_iota(jnp.int32, (b, s), 1)
        ohe_ref[...] = (iota_e == lens).astype(jnp.float32)
        oho_ref[...] = (iota_o == lens - 1).astype(jnp.float32)
        ae_ref[...] = jnp.zeros((b, ncol), jnp.float32)
        ao_ref[...] = jnp.zeros((b, s), jnp.float32)

    for bb in range(b):
        p_ref[:, bb, :] = jnp.dot(x_ref[bb], e_ref[bb],
                                  preferred_element_type=jnp.float32)

    @pl.when(i == 0)
    def _init_state():
        row0 = p_ref[0]
        iota = lax.broadcasted_iota(jnp.int32, (b, ncol), 1)
        ae_ref[...] = jnp.where(iota == 0, row0, NEG_INF)
        ao_ref[...] = jnp.where(iota[:, :s] == 0, row0[:, 1:], NEG_INF)

    skip = m_ref[...] > 0.5

    def step(t, carry):
        ae, ao = carry
        row = p_ref[t]
        lpb = row[:, :1]
        lpo = row[:, 1:]
        ao_pad = jnp.concatenate(
            [jnp.full((b, 1), NEG_INF, jnp.float32), ao], axis=1)
        m1 = jnp.maximum(ae, ao_pad)
        new_ae = m1 + jnp.log1p(jnp.exp(-jnp.abs(ae - ao_pad))) + lpb
        sk = jnp.where(skip, ao_pad[:, :s], NEG_INF)
        ae_s = ae[:, :s]
        m2 = jnp.maximum(jnp.maximum(ao, ae_s), sk)
        new_ao = (m2 + jnp.log(jnp.exp(ao - m2) + jnp.exp(ae_s - m2)
                               + jnp.exp(sk - m2)) + lpo)
        return (new_ae, new_ao)

    start_t = jnp.where(i == 0, 1, 0)
    ae_fin, ao_fin = lax.fori_loop(start_t, TC_CHUNK, step,
                                   (ae_ref[...], ao_ref[...]))
    ae_ref[...] = ae_fin
    ao_ref[...] = ao_fin

    @pl.when(i == nchunk - 1)
    def _final():
        ae = ae_ref[...]
        ao = ao_ref[...]
        ohe = ohe_ref[...] > 0.5
        oho = oho_ref[...] > 0.5
        l_last = jnp.sum(jnp.where(ohe, ae, 0.0), axis=1, keepdims=True)
        l_prev = jnp.sum(jnp.where(oho, ao, 0.0), axis=1, keepdims=True)
        mm = jnp.maximum(l_last, l_prev)
        nll = -(mm + jnp.log(jnp.exp(l_last - mm) + jnp.exp(l_prev - mm)))
        nll = jnp.where(jnp.isinf(nll) | (nll >= 1e29), 0.0, nll)
        lens = jnp.maximum(len_ref[...], 1).astype(jnp.float32)
        out_ref[...] = jnp.sum(nll / lens, keepdims=True).reshape(1, 1) * (1.0 / b)


def _ctc_call(ctc_logits, targets, lengths2d):
    b, t_total, v = ctc_logits.shape
    s = targets.shape[1]
    ncol = s + 1
    nchunk = t_total // TC_CHUNK
    return pl.pallas_call(
        functools.partial(_ctc_body, b=b, s=s, v=v, t_total=t_total),
        grid=(nchunk,),
        in_specs=[
            pl.BlockSpec((b, TC_CHUNK, v), lambda i: (0, i, 0)),
            pl.BlockSpec((b, s), lambda i: (0, 0)),
            pl.BlockSpec((b, 1), lambda i: (0, 0)),
        ],
        out_specs=pl.BlockSpec((1, 1), lambda i: (0, 0)),
        out_shape=jax.ShapeDtypeStruct((1, 1), jnp.float32),
        scratch_shapes=[
            pltpu.VMEM((b, v, ncol), jnp.float32),
            pltpu.VMEM((TC_CHUNK, b, ncol), jnp.float32),
            pltpu.VMEM((b, ncol), jnp.float32),
            pltpu.VMEM((b, s), jnp.float32),
            pltpu.VMEM((b, s), jnp.float32),
            pltpu.VMEM((b, ncol), jnp.float32),
            pltpu.VMEM((b, s), jnp.float32),
        ],
        interpret=False,
    )(ctc_logits, targets, lengths2d)


@jax.jit
def kernel(att_logits, ctc_logits, targets, target_lengths):
    b = att_logits.shape[0]
    att = _att_loss_call(att_logits, targets)[0, 0]
    ctc = _ctc_call(ctc_logits, targets, target_lengths.reshape(b, 1))[0, 0]
    return ALPHA * att + (1.0 - ALPHA) * ctc
